# baseline (speedup 1.0000x reference)
import functools

import jax
import jax.numpy as jnp
from jax import lax
from jax.experimental import pallas as pl
from jax.experimental.pallas import tpu as pltpu
from jax.experimental.pallas import tpu_sc as plsc

NDIM = 128
HID = 64
N = 10000
E = 320000
NTRASH = 10016
NC = 2
NS = 16
CHUNK = 128
ROWS_PER_TILE = 158
EPT = ROWS_PER_TILE * CHUNK
EP = EPT * NS
NPAIR = ROWS_PER_TILE // 2

WB_SZ_LO, WB_SZ_HI = 624, 640
Z_SZ_LO, Z_SZ_HI = 624, 656

R_BLK = 2000


def _dense_body(x_ref, w_ref, p1_ref, p2_ref, gh_ref):
    y = jnp.dot(x_ref[0], w_ref[0], preferred_element_type=jnp.float32)
    p1_ref[0] = y[:, :NDIM]
    p2_ref[0] = y[:, NDIM:2 * NDIM]
    gh_ref[0] = y[:, 2 * NDIM:]


def _dense(x, wcat):
    nb = N // R_BLK
    return pl.pallas_call(
        _dense_body,
        grid=(2, nb),
        in_specs=[
            pl.BlockSpec((1, R_BLK, HID), lambda c, i: (c, i, 0)),
            pl.BlockSpec((1, HID, 2 * NDIM + 3 * HID), lambda c, i: (c, 0, 0)),
        ],
        out_specs=[
            pl.BlockSpec((1, R_BLK, NDIM), lambda c, i: (c, i, 0)),
            pl.BlockSpec((1, R_BLK, NDIM), lambda c, i: (c, i, 0)),
            pl.BlockSpec((1, R_BLK, 3 * HID), lambda c, i: (c, i, 0)),
        ],
        out_shape=[
            jax.ShapeDtypeStruct((2, N, NDIM), jnp.float32),
            jax.ShapeDtypeStruct((2, N, NDIM), jnp.float32),
            jax.ShapeDtypeStruct((2, N, 3 * HID), jnp.float32),
        ],
    )(x, wcat)


def _gru_body(scat_ref, deg_ref, p2_ref, gh_ref, h_ref, wih_ref,
              msgb_ref, bih_ref, bhh_ref, o_ref):
    deg = deg_ref[0][:, 0:1]
    aggr = scat_ref[0] + deg * (p2_ref[0] + msgb_ref[0])
    gi = jnp.dot(aggr, wih_ref[0], preferred_element_type=jnp.float32)
    gi = gi + bih_ref[0]
    gh = gh_ref[0] + bhh_ref[0]
    h = h_ref[0]
    r = jax.nn.sigmoid(gi[:, :HID] + gh[:, :HID])
    z = jax.nn.sigmoid(gi[:, HID:2 * HID] + gh[:, HID:2 * HID])
    n = jnp.tanh(gi[:, 2 * HID:] + r * gh[:, 2 * HID:])
    o_ref[0] = (1.0 - z) * n + z * h


def _gru(scat, deg, p2, gh, h, wih, msgb, bih, bhh):
    nb = N // R_BLK
    out_spec = pl.BlockSpec((1, R_BLK, HID), lambda c, i: (c, i, 0))
    out_shape = jax.ShapeDtypeStruct((2, N, HID), jnp.float32)
    return pl.pallas_call(
        _gru_body,
        grid=(2, nb),
        in_specs=[
            pl.BlockSpec((1, R_BLK, NDIM), lambda c, i: (c, i, 0)),
            pl.BlockSpec((1, R_BLK, NDIM), lambda c, i: (c, i, 0)),
            pl.BlockSpec((1, R_BLK, NDIM), lambda c, i: (c, i, 0)),
            pl.BlockSpec((1, R_BLK, 3 * HID), lambda c, i: (c, i, 0)),
            pl.BlockSpec((1, R_BLK, HID), lambda c, i: (c, i, 0)),
            pl.BlockSpec((1, NDIM, 3 * HID), lambda c, i: (c, 0, 0)),
            pl.BlockSpec((1, 1, NDIM), lambda c, i: (c, 0, 0)),
            pl.BlockSpec((1, 1, 3 * HID), lambda c, i: (c, 0, 0)),
            pl.BlockSpec((1, 1, 3 * HID), lambda c, i: (c, 0, 0)),
        ],
        out_specs=out_spec,
        out_shape=out_shape,
    )(scat, deg, p2, gh, h, wih, msgb, bih, bhh)


def _gsum_body(x_ref, wf_ref, bf_ref, gw_ref, gb_ref, o1_ref, o2_ref):
    x = x_ref[0]
    outs = (o1_ref, o2_ref)
    for k in range(2):
        hv = jnp.dot(x, wf_ref[k], preferred_element_type=jnp.float32)
        hv = hv + bf_ref[k]
        gl = jnp.sum(x * gw_ref[k], axis=1, keepdims=True) + gb_ref[k]
        outs[k][0] = jnp.sum(hv * jax.nn.sigmoid(gl), axis=0, keepdims=True)


def _gsum(hc3, wf, bf, gw, gb):
    idx = hc3.shape[1]
    return pl.pallas_call(
        _gsum_body,
        grid=(hc3.shape[0],),
        in_specs=[
            pl.BlockSpec((1, idx, NDIM), lambda b: (b, 0, 0)),
            pl.BlockSpec((2, NDIM, NDIM), lambda b: (0, 0, 0)),
            pl.BlockSpec((2, 1, NDIM), lambda b: (0, 0, 0)),
            pl.BlockSpec((2, 1, NDIM), lambda b: (0, 0, 0)),
            pl.BlockSpec(memory_space=pltpu.SMEM),
        ],
        out_specs=[
            pl.BlockSpec((1, 1, NDIM), lambda b: (b, 0, 0)),
            pl.BlockSpec((1, 1, NDIM), lambda b: (b, 0, 0)),
        ],
        out_shape=[
            jax.ShapeDtypeStruct((hc3.shape[0], 1, NDIM), jnp.float32),
            jax.ShapeDtypeStruct((hc3.shape[0], 1, NDIM), jnp.float32),
        ],
    )(hc3, wf, bf, gw, gb)


@functools.cache
def _sc_mesh():
    return plsc.VectorSubcoreMesh(core_axis_name="c", subcore_axis_name="s",
                                  num_cores=NC, num_subcores=NS)


def _tile_ranges(tid, lo, hi):
    return tid * lo, lo


def _scatter_body(p_hbm, gidx_hbm, sidx_hbm, zrows_hbm, out_hbm,
                  gv, sv, rows, acc, semg0, semg1, sems0, sems1):
    cid = lax.axis_index("c")
    tid = lax.axis_index("s")
    wid = cid * NS + tid

    @pl.when(tid < NS - 1)
    def _():
        pltpu.sync_copy(zrows_hbm.at[pl.ds(tid * Z_SZ_LO, Z_SZ_LO)],
                        acc.at[pl.ds(tid * Z_SZ_LO, Z_SZ_LO)])

    @pl.when(tid == NS - 1)
    def _():
        pltpu.sync_copy(zrows_hbm.at[pl.ds((NS - 1) * Z_SZ_LO, Z_SZ_HI)],
                        acc.at[pl.ds((NS - 1) * Z_SZ_LO, Z_SZ_HI)])

    plsc.subcore_barrier()

    semg = (semg0, semg1)
    sems = (sems0, sems1)

    def load_and_gather(b, j):
        pltpu.sync_copy(gidx_hbm.at[wid, j], gv.at[b])
        pltpu.sync_copy(sidx_hbm.at[wid, j], sv.at[b])
        pltpu.make_async_copy(p_hbm.at[gv.at[b]], rows.at[b], semg[b]).start()

    def drain_gather(b):
        pltpu.make_async_copy(p_hbm.at[gv.at[b]], rows.at[b], semg[b]).wait()

    def start_scatter(b):
        pltpu.async_copy(rows.at[b], acc.at[sv.at[b]], sems[b], add=True)

    def drain_scatter(b):
        pltpu.make_async_copy(rows.at[b], acc.at[sv.at[b]], sems[b]).wait()

    load_and_gather(0, 0)
    load_and_gather(1, 1)

    def body(g, carry):
        drain_gather(0)
        start_scatter(0)
        drain_gather(1)
        start_scatter(1)

        @pl.when(g < NPAIR - 1)
        def _():
            drain_scatter(0)
            load_and_gather(0, 2 * g + 2)
            drain_scatter(1)
            load_and_gather(1, 2 * g + 3)

        return carry

    lax.fori_loop(0, NPAIR, body, 0)
    drain_scatter(0)
    drain_scatter(1)
    plsc.subcore_barrier()

    @pl.when(tid < NS - 1)
    def _():
        pltpu.sync_copy(acc.at[pl.ds(tid * WB_SZ_LO, WB_SZ_LO)],
                        out_hbm.at[pl.ds(cid * N + tid * WB_SZ_LO, WB_SZ_LO)])

    @pl.when(tid == NS - 1)
    def _():
        pltpu.sync_copy(
            acc.at[pl.ds((NS - 1) * WB_SZ_LO, WB_SZ_HI)],
            out_hbm.at[pl.ds(cid * N + (NS - 1) * WB_SZ_LO, WB_SZ_HI)])


@functools.cache
def _sc_scatter_fn(width):
    return pl.kernel(
        _scatter_body,
        out_type=jax.ShapeDtypeStruct((2 * N, width), jnp.float32),
        mesh=_sc_mesh(),
        scratch_types=[
            pltpu.VMEM((2, CHUNK), jnp.int32),
            pltpu.VMEM((2, CHUNK), jnp.int32),
            pltpu.VMEM((2, CHUNK, width), jnp.float32),
            pltpu.VMEM_SHARED((NTRASH, width), jnp.float32),
            pltpu.SemaphoreType.DMA,
            pltpu.SemaphoreType.DMA,
            pltpu.SemaphoreType.DMA,
            pltpu.SemaphoreType.DMA,
        ],
    )


def _sc_scatter(p, gidx, sidx, zrows):
    return _sc_scatter_fn(p.shape[-1])(p, gidx, sidx, zrows)


def _deg_kernel(sidx_hbm, ones_hbm, zrows_hbm, out_hbm, sv, ones_v, acc, sem):
    cid = lax.axis_index("c")
    tid = lax.axis_index("s")
    wid = cid * NS + tid

    @pl.when(tid < NS - 1)
    def _():
        pltpu.sync_copy(zrows_hbm.at[pl.ds(tid * Z_SZ_LO, Z_SZ_LO)],
                        acc.at[pl.ds(tid * Z_SZ_LO, Z_SZ_LO)])

    @pl.when(tid == NS - 1)
    def _():
        pltpu.sync_copy(zrows_hbm.at[pl.ds((NS - 1) * Z_SZ_LO, Z_SZ_HI)],
                        acc.at[pl.ds((NS - 1) * Z_SZ_LO, Z_SZ_HI)])

    pltpu.sync_copy(sidx_hbm.at[wid], sv)
    pltpu.sync_copy(ones_hbm, ones_v)
    plsc.subcore_barrier()

    def body(j, carry):
        pltpu.sync_copy(ones_v, acc.at[sv.at[j]], add=True)
        return carry

    lax.fori_loop(0, ROWS_PER_TILE, body, 0)
    plsc.subcore_barrier()

    @pl.when(tid < NS - 1)
    def _():
        pltpu.sync_copy(acc.at[pl.ds(tid * WB_SZ_LO, WB_SZ_LO)],
                        out_hbm.at[pl.ds(cid * N + tid * WB_SZ_LO, WB_SZ_LO)])

    @pl.when(tid == NS - 1)
    def _():
        pltpu.sync_copy(
            acc.at[pl.ds((NS - 1) * WB_SZ_LO, WB_SZ_HI)],
            out_hbm.at[pl.ds(cid * N + (NS - 1) * WB_SZ_LO, WB_SZ_HI)])


@functools.cache
def _sc_deg_fn():
    return pl.kernel(
        _deg_kernel,
        out_type=jax.ShapeDtypeStruct((2 * N, NDIM), jnp.float32),
        mesh=_sc_mesh(),
        scratch_types=[
            pltpu.VMEM((ROWS_PER_TILE, CHUNK), jnp.int32),
            pltpu.VMEM((CHUNK, NDIM), jnp.float32),
            pltpu.VMEM_SHARED((NTRASH, NDIM), jnp.float32),
            pltpu.SemaphoreType.DMA,
        ],
    )


def _sc_deg(*args):
    return _sc_deg_fn()(*args)


def kernel(h, edge_index, params):
    idx = h.shape[1]
    hf = h.reshape(-1, NDIM)
    hstate = jnp.stack([hf[:, :HID], hf[:, HID:]])

    src = edge_index[0]
    dst = edge_index[1]
    npad = EP - E
    padg = jnp.zeros((npad,), jnp.int32)
    pads = jnp.full((npad,), N, jnp.int32)
    gidx = jnp.concatenate([src, padg, dst + N, padg]).reshape(
        2 * NS, ROWS_PER_TILE, CHUNK)
    sidx = jnp.concatenate([dst, pads, src, pads]).reshape(
        2 * NS, ROWS_PER_TILE, CHUNK)

    zrows = jnp.zeros((NTRASH, NDIM), jnp.float32)
    ones128 = jnp.concatenate(
        [jnp.ones((CHUNK, 1), jnp.float32),
         jnp.zeros((CHUNK, NDIM - 1), jnp.float32)], axis=1)

    p = params
    wcat, wih, msgb, bih, bhh = [], [], [], [], []
    for l in range(2):
        wc, wi, mb, bi, bh = [], [], [], [], []
        for d in ("f", "b"):
            pre = d + str(l) + "_"
            mw = p[pre + "msg_W"]
            wc.append(jnp.concatenate(
                [mw[:, :HID].T, mw[:, HID:].T, p[pre + "Whh"].T], axis=1))
            wi.append(p[pre + "Wih"].T)
            mb.append(p[pre + "msg_b"][None, :])
            bi.append(p[pre + "bih"][None, :])
            bh.append(p[pre + "bhh"][None, :])
        wcat.append(jnp.stack(wc))
        wih.append(jnp.stack(wi))
        msgb.append(jnp.stack(mb))
        bih.append(jnp.stack(bi))
        bhh.append(jnp.stack(bh))

    deg = _sc_deg(sidx, ones128, zrows).reshape(2, N, NDIM)

    for l in range(2):
        p1, p2, gh = _dense(hstate, wcat[l])
        scat = _sc_scatter(p1.reshape(2 * N, NDIM), gidx, sidx,
                           zrows).reshape(2, N, NDIM)
        hstate = _gru(scat, deg, p2, gh, hstate, wih[l],
                      msgb[l], bih[l], bhh[l])

    hc = jnp.concatenate([hstate[0], hstate[1]], axis=1)
    hc3 = hc.reshape(-1, idx, NDIM)

    wf = jnp.stack([p["fm_W"].T, p["fmi_W"].T])
    bf = jnp.stack([p["fm_b"][None, :], p["fmi_b"][None, :]])
    gw = jnp.stack([p["gm_W"], p["gmi_W"]])
    gb = jnp.stack([p["gm_b"][0], p["gmi_b"][0]])
    h_g, h_g_init = _gsum(hc3, wf, bf, gw, gb)
    return (hc3, h_g.reshape(-1, NDIM), h_g_init.reshape(-1, NDIM))


# speedup vs baseline: 1.1736x; 1.1736x over previous
import functools

import jax
import jax.numpy as jnp
from jax import lax
from jax.experimental import pallas as pl
from jax.experimental.pallas import tpu as pltpu
from jax.experimental.pallas import tpu_sc as plsc

NDIM = 128
HID = 64
N = 10000
E = 320000
NTRASH = 10016
NC = 2
NS = 16
CHUNK = 128
ROWS_PER_TILE = 157
EPT = ROWS_PER_TILE * CHUNK
EP = EPT * NS

WB_SZ_LO, WB_SZ_HI = 624, 640
Z_SZ_LO, Z_SZ_HI = 624, 656

R_BLK = 2000


def _dense_body(x_ref, w_ref, p1_ref, p2_ref, gh_ref):
    y = jnp.dot(x_ref[0], w_ref[0], preferred_element_type=jnp.float32)
    p1_ref[0] = y[:, :NDIM]
    p2_ref[0] = y[:, NDIM:2 * NDIM]
    gh_ref[0] = y[:, 2 * NDIM:]


def _dense(x, wcat):
    nb = N // R_BLK
    return pl.pallas_call(
        _dense_body,
        grid=(2, nb),
        in_specs=[
            pl.BlockSpec((1, R_BLK, HID), lambda c, i: (c, i, 0)),
            pl.BlockSpec((1, HID, 2 * NDIM + 3 * HID), lambda c, i: (c, 0, 0)),
        ],
        out_specs=[
            pl.BlockSpec((1, R_BLK, NDIM), lambda c, i: (c, i, 0)),
            pl.BlockSpec((1, R_BLK, NDIM), lambda c, i: (c, i, 0)),
            pl.BlockSpec((1, R_BLK, 3 * HID), lambda c, i: (c, i, 0)),
        ],
        out_shape=[
            jax.ShapeDtypeStruct((2, N, NDIM), jnp.float32),
            jax.ShapeDtypeStruct((2, N, NDIM), jnp.float32),
            jax.ShapeDtypeStruct((2, N, 3 * HID), jnp.float32),
        ],
    )(x, wcat)


def _gru_body(scat_ref, deg_ref, p2_ref, gh_ref, h_ref, wih_ref,
              msgb_ref, bih_ref, bhh_ref, o_ref):
    deg = deg_ref[0][:, 0:1]
    aggr = scat_ref[0] + deg * (p2_ref[0] + msgb_ref[0])
    gi = jnp.dot(aggr, wih_ref[0], preferred_element_type=jnp.float32)
    gi = gi + bih_ref[0]
    gh = gh_ref[0] + bhh_ref[0]
    h = h_ref[0]
    r = jax.nn.sigmoid(gi[:, :HID] + gh[:, :HID])
    z = jax.nn.sigmoid(gi[:, HID:2 * HID] + gh[:, HID:2 * HID])
    n = jnp.tanh(gi[:, 2 * HID:] + r * gh[:, 2 * HID:])
    o_ref[0] = (1.0 - z) * n + z * h


def _gru(scat, deg, p2, gh, h, wih, msgb, bih, bhh):
    nb = N // R_BLK
    out_spec = pl.BlockSpec((1, R_BLK, HID), lambda c, i: (c, i, 0))
    out_shape = jax.ShapeDtypeStruct((2, N, HID), jnp.float32)
    return pl.pallas_call(
        _gru_body,
        grid=(2, nb),
        in_specs=[
            pl.BlockSpec((1, R_BLK, NDIM), lambda c, i: (c, i, 0)),
            pl.BlockSpec((1, R_BLK, NDIM), lambda c, i: (c, i, 0)),
            pl.BlockSpec((1, R_BLK, NDIM), lambda c, i: (c, i, 0)),
            pl.BlockSpec((1, R_BLK, 3 * HID), lambda c, i: (c, i, 0)),
            pl.BlockSpec((1, R_BLK, HID), lambda c, i: (c, i, 0)),
            pl.BlockSpec((1, NDIM, 3 * HID), lambda c, i: (c, 0, 0)),
            pl.BlockSpec((1, 1, NDIM), lambda c, i: (c, 0, 0)),
            pl.BlockSpec((1, 1, 3 * HID), lambda c, i: (c, 0, 0)),
            pl.BlockSpec((1, 1, 3 * HID), lambda c, i: (c, 0, 0)),
        ],
        out_specs=out_spec,
        out_shape=out_shape,
    )(scat, deg, p2, gh, h, wih, msgb, bih, bhh)


def _gsum_body(x_ref, wf_ref, bf_ref, gw_ref, gb_ref, o1_ref, o2_ref):
    x = x_ref[0]
    outs = (o1_ref, o2_ref)
    for k in range(2):
        hv = jnp.dot(x, wf_ref[k], preferred_element_type=jnp.float32)
        hv = hv + bf_ref[k]
        gl = jnp.sum(x * gw_ref[k], axis=1, keepdims=True) + gb_ref[k]
        outs[k][0] = jnp.sum(hv * jax.nn.sigmoid(gl), axis=0, keepdims=True)


def _gsum(hc3, wf, bf, gw, gb):
    idx = hc3.shape[1]
    return pl.pallas_call(
        _gsum_body,
        grid=(hc3.shape[0],),
        in_specs=[
            pl.BlockSpec((1, idx, NDIM), lambda b: (b, 0, 0)),
            pl.BlockSpec((2, NDIM, NDIM), lambda b: (0, 0, 0)),
            pl.BlockSpec((2, 1, NDIM), lambda b: (0, 0, 0)),
            pl.BlockSpec((2, 1, NDIM), lambda b: (0, 0, 0)),
            pl.BlockSpec(memory_space=pltpu.SMEM),
        ],
        out_specs=[
            pl.BlockSpec((1, 1, NDIM), lambda b: (b, 0, 0)),
            pl.BlockSpec((1, 1, NDIM), lambda b: (b, 0, 0)),
        ],
        out_shape=[
            jax.ShapeDtypeStruct((hc3.shape[0], 1, NDIM), jnp.float32),
            jax.ShapeDtypeStruct((hc3.shape[0], 1, NDIM), jnp.float32),
        ],
    )(hc3, wf, bf, gw, gb)


@functools.cache
def _sc_mesh():
    return plsc.VectorSubcoreMesh(core_axis_name="c", subcore_axis_name="s",
                                  num_cores=NC, num_subcores=NS)


def _tile_ranges(tid, lo, hi):
    return tid * lo, lo


def _scatter_body(p_hbm, gidx_hbm, sidx_hbm, zrows_hbm, out_hbm,
                  gv, sv, rows, acc, sem, semi):
    cid = lax.axis_index("c")
    tid = lax.axis_index("s")
    wid = cid * NS + tid

    @pl.when(tid < NS - 1)
    def _():
        pltpu.sync_copy(zrows_hbm.at[pl.ds(tid * Z_SZ_LO, Z_SZ_LO)],
                        acc.at[pl.ds(tid * Z_SZ_LO, Z_SZ_LO)])

    @pl.when(tid == NS - 1)
    def _():
        pltpu.sync_copy(zrows_hbm.at[pl.ds((NS - 1) * Z_SZ_LO, Z_SZ_HI)],
                        acc.at[pl.ds((NS - 1) * Z_SZ_LO, Z_SZ_HI)])

    pltpu.sync_copy(sidx_hbm.at[wid], sv)
    pltpu.sync_copy(gidx_hbm.at[wid, 0], gv.at[0])
    plsc.subcore_barrier()

    def body(j, carry):
        b = lax.rem(j, 2)
        bn = 1 - b

        @pl.when(j < ROWS_PER_TILE - 1)
        def _():
            pltpu.async_copy(gidx_hbm.at[wid, j + 1], gv.at[bn], semi)

        pltpu.async_copy(p_hbm.at[gv.at[b]], rows, sem).wait()
        pltpu.sync_copy(rows, acc.at[sv.at[j]], add=True)

        @pl.when(j < ROWS_PER_TILE - 1)
        def _():
            pltpu.make_async_copy(gidx_hbm.at[wid, j + 1], gv.at[bn],
                                  semi).wait()

        return carry

    lax.fori_loop(0, ROWS_PER_TILE, body, 0)
    plsc.subcore_barrier()

    @pl.when(tid < NS - 1)
    def _():
        pltpu.sync_copy(acc.at[pl.ds(tid * WB_SZ_LO, WB_SZ_LO)],
                        out_hbm.at[pl.ds(cid * N + tid * WB_SZ_LO, WB_SZ_LO)])

    @pl.when(tid == NS - 1)
    def _():
        pltpu.sync_copy(
            acc.at[pl.ds((NS - 1) * WB_SZ_LO, WB_SZ_HI)],
            out_hbm.at[pl.ds(cid * N + (NS - 1) * WB_SZ_LO, WB_SZ_HI)])


@functools.cache
def _sc_scatter_fn(width):
    return pl.kernel(
        _scatter_body,
        out_type=jax.ShapeDtypeStruct((2 * N, width), jnp.float32),
        mesh=_sc_mesh(),
        scratch_types=[
            pltpu.VMEM((2, CHUNK), jnp.int32),
            pltpu.VMEM((ROWS_PER_TILE, CHUNK), jnp.int32),
            pltpu.VMEM((CHUNK, width), jnp.float32),
            pltpu.VMEM_SHARED((NTRASH, width), jnp.float32),
            pltpu.SemaphoreType.DMA,
            pltpu.SemaphoreType.DMA,
        ],
    )


def _sc_scatter(p, gidx, sidx, zrows):
    return _sc_scatter_fn(p.shape[-1])(p, gidx, sidx, zrows)


def _deg_kernel(sidx_hbm, ones_hbm, zrows_hbm, out_hbm, sv, ones_v, acc, sem):
    cid = lax.axis_index("c")
    tid = lax.axis_index("s")
    wid = cid * NS + tid

    @pl.when(tid < NS - 1)
    def _():
        pltpu.sync_copy(zrows_hbm.at[pl.ds(tid * Z_SZ_LO, Z_SZ_LO)],
                        acc.at[pl.ds(tid * Z_SZ_LO, Z_SZ_LO)])

    @pl.when(tid == NS - 1)
    def _():
        pltpu.sync_copy(zrows_hbm.at[pl.ds((NS - 1) * Z_SZ_LO, Z_SZ_HI)],
                        acc.at[pl.ds((NS - 1) * Z_SZ_LO, Z_SZ_HI)])

    pltpu.sync_copy(sidx_hbm.at[wid], sv)
    pltpu.sync_copy(ones_hbm, ones_v)
    plsc.subcore_barrier()

    def body(j, carry):
        pltpu.sync_copy(ones_v, acc.at[sv.at[j]], add=True)
        return carry

    lax.fori_loop(0, ROWS_PER_TILE, body, 0)
    plsc.subcore_barrier()

    @pl.when(tid < NS - 1)
    def _():
        pltpu.sync_copy(acc.at[pl.ds(tid * WB_SZ_LO, WB_SZ_LO)],
                        out_hbm.at[pl.ds(cid * N + tid * WB_SZ_LO, WB_SZ_LO)])

    @pl.when(tid == NS - 1)
    def _():
        pltpu.sync_copy(
            acc.at[pl.ds((NS - 1) * WB_SZ_LO, WB_SZ_HI)],
            out_hbm.at[pl.ds(cid * N + (NS - 1) * WB_SZ_LO, WB_SZ_HI)])


@functools.cache
def _sc_deg_fn():
    return pl.kernel(
        _deg_kernel,
        out_type=jax.ShapeDtypeStruct((2 * N, NDIM), jnp.float32),
        mesh=_sc_mesh(),
        scratch_types=[
            pltpu.VMEM((ROWS_PER_TILE, CHUNK), jnp.int32),
            pltpu.VMEM((CHUNK, NDIM), jnp.float32),
            pltpu.VMEM_SHARED((NTRASH, NDIM), jnp.float32),
            pltpu.SemaphoreType.DMA,
        ],
    )


def _sc_deg(*args):
    return _sc_deg_fn()(*args)


def kernel(h, edge_index, params):
    idx = h.shape[1]
    hf = h.reshape(-1, NDIM)
    hstate = jnp.stack([hf[:, :HID], hf[:, HID:]])

    src = edge_index[0]
    dst = edge_index[1]
    npad = EP - E
    padg = jnp.zeros((npad,), jnp.int32)
    pads = jnp.full((npad,), N, jnp.int32)
    gidx = jnp.concatenate([src, padg, dst + N, padg]).reshape(
        2 * NS, ROWS_PER_TILE, CHUNK)
    sidx = jnp.concatenate([dst, pads, src, pads]).reshape(
        2 * NS, ROWS_PER_TILE, CHUNK)

    zrows = jnp.zeros((NTRASH, NDIM), jnp.float32)
    ones128 = jnp.concatenate(
        [jnp.ones((CHUNK, 1), jnp.float32),
         jnp.zeros((CHUNK, NDIM - 1), jnp.float32)], axis=1)

    p = params
    wcat, wih, msgb, bih, bhh = [], [], [], [], []
    for l in range(2):
        wc, wi, mb, bi, bh = [], [], [], [], []
        for d in ("f", "b"):
            pre = d + str(l) + "_"
            mw = p[pre + "msg_W"]
            wc.append(jnp.concatenate(
                [mw[:, :HID].T, mw[:, HID:].T, p[pre + "Whh"].T], axis=1))
            wi.append(p[pre + "Wih"].T)
            mb.append(p[pre + "msg_b"][None, :])
            bi.append(p[pre + "bih"][None, :])
            bh.append(p[pre + "bhh"][None, :])
        wcat.append(jnp.stack(wc))
        wih.append(jnp.stack(wi))
        msgb.append(jnp.stack(mb))
        bih.append(jnp.stack(bi))
        bhh.append(jnp.stack(bh))

    deg = _sc_deg(sidx, ones128, zrows).reshape(2, N, NDIM)

    for l in range(2):
        p1, p2, gh = _dense(hstate, wcat[l])
        scat = _sc_scatter(p1.reshape(2 * N, NDIM), gidx, sidx,
                           zrows).reshape(2, N, NDIM)
        hstate = _gru(scat, deg, p2, gh, hstate, wih[l],
                      msgb[l], bih[l], bhh[l])

    hc = jnp.concatenate([hstate[0], hstate[1]], axis=1)
    hc3 = hc.reshape(-1, idx, NDIM)

    wf = jnp.stack([p["fm_W"].T, p["fmi_W"].T])
    bf = jnp.stack([p["fm_b"][None, :], p["fmi_b"][None, :]])
    gw = jnp.stack([p["gm_W"], p["gmi_W"]])
    gb = jnp.stack([p["gm_b"][0], p["gmi_b"][0]])
    h_g, h_g_init = _gsum(hc3, wf, bf, gw, gb)
    return (hc3, h_g.reshape(-1, NDIM), h_g_init.reshape(-1, NDIM))
